# baseline (speedup 1.0000x reference)
import functools

import jax
import jax.numpy as jnp
from jax import lax
from jax.experimental import pallas as pl
from jax.experimental.pallas import tpu as pltpu
from jax.experimental.pallas import tpu_sc as plsc

VOCAB = 1000
EMB = 16
B, L = 4096, 20
N = B * L

_NC, _NS = 2, 16
_NW = _NC * _NS
_B_PER_W = B // _NW

_SC_MESH = plsc.VectorSubcoreMesh(core_axis_name="c", subcore_axis_name="s")


@functools.partial(
    pl.kernel,
    out_type=jax.ShapeDtypeStruct((L, EMB, B), jnp.float32),
    mesh=_SC_MESH,
    scratch_types=[
        pltpu.VMEM((L, _B_PER_W), jnp.int32),
        pltpu.VMEM((VOCAB, EMB), jnp.float32),
        pltpu.VMEM((L, EMB, _B_PER_W), jnp.float32),
        pltpu.SemaphoreType.DMA,
    ],
    compiler_params=pltpu.CompilerParams(
        use_tc_tiling_on_sc=False, needs_layout_passes=False
    ),
)
def _sc_gather_emb(table_hbm, xt_hbm, emb_hbm, idx_v, tab_v, trans_v, wsem):
    wid = lax.axis_index("s") * _NC + lax.axis_index("c")
    b0 = wid * _B_PER_W
    pltpu.sync_copy(xt_hbm.at[:, pl.ds(b0, _B_PER_W)], idx_v)
    pltpu.sync_copy(table_hbm, tab_v)

    def body(l, _):
        for k in range(_B_PER_W // 16):
            idxv = idx_v[l, pl.ds(k * 16, 16)]
            for e in range(EMB):
                col = jnp.full((16,), e, jnp.int32)
                vals = plsc.load_gather(tab_v, [idxv, col])
                trans_v[l, e, pl.ds(k * 16, 16)] = vals
        return 0

    lax.fori_loop(0, L, body, 0)
    pltpu.async_copy(
        trans_v, emb_hbm.at[:, :, pl.ds(b0, _B_PER_W)], wsem
    ).wait()


_BT = 1024


def _proj_body(w_ref, b_ref, emb_ref, out_ref):
    e = emb_ref[0]
    m = lax.dot_general(
        w_ref[...], e, (((0,), (0,)), ((), ())),
        preferred_element_type=jnp.float32,
    )
    out_ref[0] = m + b_ref[...]


def _tc_project(W, b2, emb3):
    grid = (L, B // _BT)
    return pl.pallas_call(
        _proj_body,
        grid=grid,
        in_specs=[
            pl.BlockSpec((EMB, VOCAB), lambda l, j: (0, 0)),
            pl.BlockSpec((VOCAB, 1), lambda l, j: (0, 0)),
            pl.BlockSpec((1, EMB, _BT), lambda l, j: (l, 0, j)),
        ],
        out_specs=pl.BlockSpec((1, VOCAB, _BT), lambda l, j: (l, 0, j)),
        out_shape=jax.ShapeDtypeStruct((L, VOCAB, B), jnp.float32),
    )(W, b2, emb3)


def kernel(x, table, W, b):
    xt = x.astype(jnp.int32).T
    emb3 = _sc_gather_emb(table, xt)
    outT = _tc_project(W, b.reshape(VOCAB, 1), emb3)
    return jnp.transpose(outT, (2, 0, 1))


# speedup vs baseline: 1.0010x; 1.0010x over previous
import functools

import jax
import jax.numpy as jnp
from jax import lax
from jax.experimental import pallas as pl
from jax.experimental.pallas import tpu as pltpu
from jax.experimental.pallas import tpu_sc as plsc

VOCAB = 1000
EMB = 16
B, L = 4096, 20
N = B * L

_NC, _NS = 2, 16
_NW = _NC * _NS
_B_PER_W = B // _NW

_SC_MESH = plsc.VectorSubcoreMesh(core_axis_name="c", subcore_axis_name="s")


@functools.partial(
    pl.kernel,
    out_type=jax.ShapeDtypeStruct((L, EMB, B), jnp.float32),
    mesh=_SC_MESH,
    scratch_types=[
        pltpu.VMEM((L, _B_PER_W), jnp.int32),
        pltpu.VMEM((VOCAB, EMB), jnp.float32),
        pltpu.VMEM((L, EMB, _B_PER_W), jnp.float32),
        pltpu.SemaphoreType.DMA,
    ],
    compiler_params=pltpu.CompilerParams(
        use_tc_tiling_on_sc=False,
        needs_layout_passes=False,
        disable_bounds_checks=True,
    ),
)
def _sc_gather_emb(table_hbm, xt_hbm, emb_hbm, idx_v, tab_v, trans_v, wsem):
    wid = lax.axis_index("s") * _NC + lax.axis_index("c")
    b0 = wid * _B_PER_W
    pltpu.sync_copy(xt_hbm.at[:, pl.ds(b0, _B_PER_W)], idx_v)
    pltpu.sync_copy(table_hbm, tab_v)

    def body(l, _):
        for k in range(_B_PER_W // 16):
            idxv = idx_v[l, pl.ds(k * 16, 16)]
            for e in range(EMB):
                col = jnp.full((16,), e, jnp.int32)
                vals = plsc.load_gather(tab_v, [idxv, col])
                trans_v[l, e, pl.ds(k * 16, 16)] = vals
        return 0

    lax.fori_loop(0, L, body, 0)
    pltpu.async_copy(
        trans_v, emb_hbm.at[:, :, pl.ds(b0, _B_PER_W)], wsem
    ).wait()


_BT = 1024


def _proj_body(w_ref, b_ref, emb_ref, out_ref):
    e = emb_ref[0]
    m = lax.dot_general(
        w_ref[...], e, (((0,), (0,)), ((), ())),
        preferred_element_type=jnp.float32,
    )
    out_ref[0] = m + b_ref[...]


def _tc_project(W, b2, emb3):
    grid = (L, B // _BT)
    return pl.pallas_call(
        _proj_body,
        grid=grid,
        in_specs=[
            pl.BlockSpec((EMB, VOCAB), lambda l, j: (0, 0)),
            pl.BlockSpec((VOCAB, 1), lambda l, j: (0, 0)),
            pl.BlockSpec((1, EMB, _BT), lambda l, j: (l, 0, j)),
        ],
        out_specs=pl.BlockSpec((1, VOCAB, _BT), lambda l, j: (l, 0, j)),
        out_shape=jax.ShapeDtypeStruct((L, VOCAB, B), jnp.float32),
    )(W, b2, emb3)


def kernel(x, table, W, b):
    xt = x.astype(jnp.int32).T
    emb3 = _sc_gather_emb(table, xt)
    outT = _tc_project(W, b.reshape(VOCAB, 1), emb3)
    return jnp.transpose(outT, (2, 0, 1))


# speedup vs baseline: 1.0988x; 1.0978x over previous
import functools

import jax
import jax.numpy as jnp
from jax import lax
from jax.experimental import pallas as pl
from jax.experimental.pallas import tpu as pltpu
from jax.experimental.pallas import tpu_sc as plsc

VOCAB = 1000
EMB = 16
B, L = 4096, 20
N = B * L

_NC, _NS = 2, 16
_NW = _NC * _NS
_B_PER_W = B // _NW

_SC_MESH = plsc.VectorSubcoreMesh(core_axis_name="c", subcore_axis_name="s")


@functools.partial(
    pl.kernel,
    out_type=jax.ShapeDtypeStruct((L, EMB, B), jnp.float32),
    mesh=_SC_MESH,
    scratch_types=[
        pltpu.VMEM((L, _B_PER_W), jnp.int32),
        pltpu.VMEM((VOCAB, EMB), jnp.float32),
        pltpu.VMEM((L, EMB, _B_PER_W), jnp.float32),
        pltpu.SemaphoreType.DMA,
    ],
    compiler_params=pltpu.CompilerParams(
        use_tc_tiling_on_sc=False,
        needs_layout_passes=False,
        disable_bounds_checks=True,
    ),
)
def _sc_gather_emb(table_hbm, xt_hbm, emb_hbm, idx_v, tab_v, trans_v, wsem):
    wid = lax.axis_index("s") * _NC + lax.axis_index("c")
    b0 = wid * _B_PER_W
    pltpu.sync_copy(xt_hbm.at[:, pl.ds(b0, _B_PER_W)], idx_v)
    pltpu.sync_copy(table_hbm, tab_v)

    def body(l, _):
        for k in range(_B_PER_W // 16):
            idxv = idx_v[l, pl.ds(k * 16, 16)]
            for e in range(EMB):
                col = jnp.full((16,), e, jnp.int32)
                vals = plsc.load_gather(tab_v, [idxv, col])
                trans_v[l, e, pl.ds(k * 16, 16)] = vals
        return 0

    lax.fori_loop(0, L, body, 0)
    pltpu.async_copy(
        trans_v, emb_hbm.at[:, :, pl.ds(b0, _B_PER_W)], wsem
    ).wait()


_BT = 2048


def _proj_body(w_ref, b_ref, emb_ref, out_ref):
    e = emb_ref[0]
    m = lax.dot_general(
        w_ref[...], e, (((0,), (0,)), ((), ())),
        preferred_element_type=jnp.float32,
    )
    out_ref[0] = m + b_ref[...]


def _tc_project(W, b2, emb3):
    grid = (L, B // _BT)
    return pl.pallas_call(
        _proj_body,
        grid=grid,
        in_specs=[
            pl.BlockSpec((EMB, VOCAB), lambda l, j: (0, 0)),
            pl.BlockSpec((VOCAB, 1), lambda l, j: (0, 0)),
            pl.BlockSpec((1, EMB, _BT), lambda l, j: (l, 0, j)),
        ],
        out_specs=pl.BlockSpec((1, VOCAB, _BT), lambda l, j: (l, 0, j)),
        out_shape=jax.ShapeDtypeStruct((L, VOCAB, B), jnp.float32),
    )(W, b2, emb3)


def kernel(x, table, W, b):
    xt = x.astype(jnp.int32).T
    emb3 = _sc_gather_emb(table, xt)
    outT = _tc_project(W, b.reshape(VOCAB, 1), emb3)
    return jnp.transpose(outT, (2, 0, 1))
